# Initial kernel scaffold; baseline (speedup 1.0000x reference)
import functools

import jax
import jax.numpy as jnp
from jax import lax
from jax.experimental import pallas as pl
from jax.experimental.pallas import tpu as pltpu
from jax.experimental.pallas import tpu_sc as plsc

D = 128
LANES = 16
N_SUB = 16
N_CORES = 2
CHUNK = 128
SEG_PAD = 10240
ROWS_PER_TILE = SEG_PAD // N_SUB
ROW_CHUNKS = ROWS_PER_TILE // CHUNK
DUMP_ROW = 10016


def _sc_segment_partial(x_func, s1, d1, s2, d2, per_tile):
    n_chunks = per_tile // CHUNK
    mesh = plsc.VectorSubcoreMesh(core_axis_name="c", subcore_axis_name="s")

    @functools.partial(
        pl.kernel,
        mesh=mesh,
        out_type=jax.ShapeDtypeStruct((N_CORES, SEG_PAD, D), jnp.float32),
        scratch_types=[
            pltpu.VMEM((CHUNK,), jnp.int32),
            pltpu.VMEM((CHUNK,), jnp.int32),
            pltpu.VMEM((CHUNK, D), jnp.float32),
            pltpu.VMEM((CHUNK, LANES), jnp.float32),
            pltpu.VMEM_SHARED((SEG_PAD, D), jnp.float32),
            pltpu.VMEM_SHARED((SEG_PAD, LANES), jnp.float32),
            pltpu.SemaphoreType.DMA,
        ],
    )
    def k(x_hbm, s1_hbm, d1_hbm, s2_hbm, d2_hbm, y_hbm,
          idx_s, idx_d, rows, ones, acc, cnt, sem):
        c = lax.axis_index("c")
        s = lax.axis_index("s")
        row_base = s * ROWS_PER_TILE
        edge_base = s * per_tile

        zero16 = jnp.zeros((LANES,), jnp.float32)
        one16 = jnp.ones((LANES,), jnp.float32)

        def zrow(r, carry):
            for j in range(D // LANES):
                rows[r, pl.ds(j * LANES, LANES)] = zero16
            return carry

        lax.fori_loop(0, CHUNK, zrow, 0)

        def zone(r, carry):
            ones[r, :] = zero16
            return carry

        lax.fori_loop(0, CHUNK, zone, 0)

        for kk in range(ROW_CHUNKS):
            pltpu.sync_copy(rows, acc.at[pl.ds(row_base + kk * CHUNK, CHUNK)])
            pltpu.sync_copy(ones, cnt.at[pl.ds(row_base + kk * CHUNK, CHUNK)])

        def fone(r, carry):
            ones[r, :] = one16
            return carry

        lax.fori_loop(0, CHUNK, fone, 0)

        plsc.subcore_barrier()

        def edge_loop(src_hbm, dst_hbm):
            def step(kk, carry):
                base = edge_base + kk * CHUNK
                pltpu.sync_copy(src_hbm.at[pl.ds(base, CHUNK)], idx_s)
                pltpu.sync_copy(dst_hbm.at[pl.ds(base, CHUNK)], idx_d)
                pltpu.async_copy(x_hbm.at[idx_s], rows, sem).wait()
                pltpu.sync_copy(rows, acc.at[idx_d], add=True)
                pltpu.sync_copy(ones, cnt.at[idx_d], add=True)
                return carry

            lax.fori_loop(0, n_chunks, step, 0)

        @pl.when(c == 0)
        def _():
            edge_loop(s1_hbm, d1_hbm)

        @pl.when(c == 1)
        def _():
            edge_loop(s2_hbm, d2_hbm)

        plsc.subcore_barrier()

        def divide_phase(etype):
            for kk in range(ROW_CHUNKS):
                rb = row_base + kk * CHUNK
                pltpu.sync_copy(acc.at[pl.ds(rb, CHUNK)], rows)
                pltpu.sync_copy(cnt.at[pl.ds(rb, CHUNK)], ones)

                def drow(r, carry):
                    scale = 0.5 / jnp.maximum(ones[r, :], 1.0)
                    for j in range(D // LANES):
                        sl = pl.ds(j * LANES, LANES)
                        rows[r, sl] = rows[r, sl] * scale
                    return carry

                lax.fori_loop(0, CHUNK, drow, 0)
                pltpu.sync_copy(rows, y_hbm.at[etype, pl.ds(rb, CHUNK)])

        @pl.when(c == 0)
        def _():
            divide_phase(0)

        @pl.when(c == 1)
        def _():
            divide_phase(1)

    return k(x_func, s1, d1, s2, d2)


def _combine_body(y_ref, x_ref, o_ref):
    o_ref[0, :, :] = y_ref[0, :, :] + y_ref[1, :, :]
    o_ref[1, :, :] = x_ref[:, :]


def _combine(y, x_func, n_seg):
    blk = 1000
    grid = n_seg // blk
    return pl.pallas_call(
        _combine_body,
        grid=(grid,),
        in_specs=[
            pl.BlockSpec((N_CORES, blk, D), lambda i: (0, i, 0)),
            pl.BlockSpec((blk, D), lambda i: (i, 0)),
        ],
        out_specs=pl.BlockSpec((N_CORES, blk, D), lambda i: (0, i, 0)),
        out_shape=jax.ShapeDtypeStruct((N_CORES, n_seg, D), jnp.float32),
    )(y, x_func)


def kernel(x_vul, x_func, edge_index_1, edge_index_2):
    n_seg = x_vul.shape[0]
    n_edges = edge_index_1.shape[1]
    per_tile = -(-n_edges // (N_SUB * CHUNK)) * CHUNK
    e_pad = per_tile * N_SUB
    npad = e_pad - n_edges

    pad_src = jnp.zeros((npad,), jnp.int32)
    pad_dst = jnp.full((npad,), DUMP_ROW, jnp.int32)
    s1 = jnp.concatenate([edge_index_1[0].astype(jnp.int32), pad_src])
    d1 = jnp.concatenate([edge_index_1[1].astype(jnp.int32), pad_dst])
    s2 = jnp.concatenate([edge_index_2[0].astype(jnp.int32), pad_src])
    d2 = jnp.concatenate([edge_index_2[1].astype(jnp.int32), pad_dst])

    y = _sc_segment_partial(x_func, s1, d1, s2, d2, per_tile)
    return _combine(y, x_func, n_seg)


# speedup vs baseline: 3.5929x; 3.5929x over previous
import functools

import jax
import jax.numpy as jnp
from jax import lax
from jax.experimental import pallas as pl
from jax.experimental.pallas import tpu as pltpu
from jax.experimental.pallas import tpu_sc as plsc

D = 128
LANES = 16
N_SUB = 16
N_CORES = 2
CHUNK = 128
SEG_PAD = 10240
ROWS_PER_TILE = SEG_PAD // N_SUB
ROW_CHUNKS = ROWS_PER_TILE // CHUNK
DUMP_ROW = 10016


def _sc_segment_sums(x_func, s1, d1, s2, d2, per_tile):
    n_chunks = per_tile // CHUNK
    mesh = plsc.VectorSubcoreMesh(core_axis_name="c", subcore_axis_name="s")

    @functools.partial(
        pl.kernel,
        mesh=mesh,
        out_type=jax.ShapeDtypeStruct((N_CORES, SEG_PAD, D), jnp.float32),
        scratch_types=[
            pltpu.VMEM((CHUNK,), jnp.int32),
            pltpu.VMEM((CHUNK,), jnp.int32),
            pltpu.VMEM((CHUNK, D), jnp.float32),
            pltpu.VMEM((ROWS_PER_TILE // 8, D), jnp.float32),
            pltpu.VMEM_SHARED((SEG_PAD, D), jnp.float32),
            pltpu.SemaphoreType.DMA,
        ],
    )
    def k(x_hbm, s1_hbm, d1_hbm, s2_hbm, d2_hbm, y_hbm,
          idx_s, idx_d, rows, compact, acc, sem):
        cid = lax.axis_index("c")
        sid = lax.axis_index("s")
        row_base = sid * ROWS_PER_TILE
        edge_base = sid * per_tile

        zero16 = jnp.zeros((LANES,), jnp.float32)
        one16 = jnp.ones((LANES,), jnp.float32)
        iota16 = lax.iota(jnp.int32, 16)

        def fill_idx(rb):
            for i in range(CHUNK // LANES):
                idx_d[pl.ds(i * LANES, LANES)] = rb + i * LANES + iota16

        def fill_buf(buf, val):
            def body(r, carry):
                for j in range(D // LANES):
                    buf[r, pl.ds(j * LANES, LANES)] = val
                return carry

            lax.fori_loop(0, CHUNK, body, 0)

        fill_buf(rows, zero16)

        for kk in range(ROW_CHUNKS):
            fill_idx(row_base + kk * CHUNK)
            pltpu.sync_copy(rows, acc.at[idx_d])

        plsc.subcore_barrier()

        def pick_etype(fn):
            @pl.when(cid == 0)
            def _():
                fn(0)

            @pl.when(cid == 1)
            def _():
                fn(1)

        def count_pass(etype):
            dst_hbm = d1_hbm if etype == 0 else d2_hbm

            def step(kk, carry):
                base = edge_base + kk * CHUNK
                pltpu.sync_copy(dst_hbm.at[pl.ds(base, CHUNK)], idx_d)
                pltpu.sync_copy(rows, acc.at[idx_d], add=True)
                return carry

            lax.fori_loop(0, n_chunks, step, 0)

        fill_buf(rows, one16)
        pick_etype(count_pass)

        plsc.subcore_barrier()

        def dchunk_cnt(kk, carry):
            fill_idx(row_base + kk * CHUNK)
            pltpu.sync_copy(acc.at[idx_d], rows)
            for r in range(CHUNK):
                compact[kk * (CHUNK // 8) + r // 8,
                        pl.ds((r % 8) * LANES, LANES)] = rows[r, pl.ds(0, LANES)]
            fill_buf(rows, zero16)
            pltpu.sync_copy(rows, acc.at[idx_d])
            return carry

        lax.fori_loop(0, ROW_CHUNKS, dchunk_cnt, 0)

        plsc.subcore_barrier()

        def sum_pass(etype):
            src_hbm = s1_hbm if etype == 0 else s2_hbm
            dst_hbm = d1_hbm if etype == 0 else d2_hbm

            def step(kk, carry):
                base = edge_base + kk * CHUNK
                pltpu.sync_copy(src_hbm.at[pl.ds(base, CHUNK)], idx_s)
                pltpu.sync_copy(dst_hbm.at[pl.ds(base, CHUNK)], idx_d)
                pltpu.async_copy(x_hbm.at[idx_s], rows, sem).wait()
                pltpu.sync_copy(rows, acc.at[idx_d], add=True)
                return carry

            lax.fori_loop(0, n_chunks, step, 0)

        pick_etype(sum_pass)

        plsc.subcore_barrier()

        def writeback(etype):
            def dchunk(kk, carry):
                rb = row_base + kk * CHUNK
                fill_idx(rb)
                pltpu.sync_copy(acc.at[idx_d], rows)
                for r in range(CHUNK):
                    cnt16 = compact[kk * (CHUNK // 8) + r // 8,
                                    pl.ds((r % 8) * LANES, LANES)]
                    scale = 0.5 / jnp.maximum(cnt16, 1.0)
                    for j in range(D // LANES):
                        sl = pl.ds(j * LANES, LANES)
                        rows[r, sl] = rows[r, sl] * scale
                pltpu.sync_copy(rows, y_hbm.at[etype, pl.ds(rb, CHUNK)])
                return carry

            lax.fori_loop(0, ROW_CHUNKS, dchunk, 0)

        pick_etype(writeback)

    return k(x_func, s1, d1, s2, d2)


def _combine_body(y_ref, x_ref, o_ref):
    o_ref[0, :, :] = y_ref[0, :, :] + y_ref[1, :, :]
    o_ref[1, :, :] = x_ref[:, :]


def _combine(y, x_pad):
    blk = 1024
    grid = SEG_PAD // blk
    return pl.pallas_call(
        _combine_body,
        grid=(grid,),
        in_specs=[
            pl.BlockSpec((N_CORES, blk, D), lambda i: (0, i, 0)),
            pl.BlockSpec((blk, D), lambda i: (i, 0)),
        ],
        out_specs=pl.BlockSpec((N_CORES, blk, D), lambda i: (0, i, 0)),
        out_shape=jax.ShapeDtypeStruct((N_CORES, SEG_PAD, D), jnp.float32),
    )(y, x_pad)


def kernel(x_vul, x_func, edge_index_1, edge_index_2):
    n_seg = x_vul.shape[0]
    n_edges = edge_index_1.shape[1]
    per_tile = -(-n_edges // (N_SUB * CHUNK)) * CHUNK
    e_pad = per_tile * N_SUB
    npad = e_pad - n_edges

    pad_src = jnp.zeros((npad,), jnp.int32)
    pad_dst = jnp.full((npad,), DUMP_ROW, jnp.int32)
    s1 = jnp.concatenate([edge_index_1[0].astype(jnp.int32), pad_src])
    d1 = jnp.concatenate([edge_index_1[1].astype(jnp.int32), pad_dst])
    s2 = jnp.concatenate([edge_index_2[0].astype(jnp.int32), pad_src])
    d2 = jnp.concatenate([edge_index_2[1].astype(jnp.int32), pad_dst])

    y = _sc_segment_sums(x_func, s1, d1, s2, d2, per_tile)
    x_pad = jnp.concatenate(
        [x_func, jnp.zeros((SEG_PAD - x_func.shape[0], D), jnp.float32)])
    out = _combine(y, x_pad)
    return out[:, :n_seg, :]


# speedup vs baseline: 6.4382x; 1.7920x over previous
import functools

import jax
import jax.numpy as jnp
from jax import lax
from jax.experimental import pallas as pl
from jax.experimental.pallas import tpu as pltpu
from jax.experimental.pallas import tpu_sc as plsc

D = 128
LANES = 16
N_SUB = 16
N_CORES = 2
CHUNK = 128
SEG_PAD = 10240
ROWS_PER_TILE = SEG_PAD // N_SUB
ROW_CHUNKS = ROWS_PER_TILE // CHUNK


def _sc_segment_partials(x_func, s1, d1, s2, d2, per_tile):
    n_full = per_tile // CHUNK
    n_pair = n_full // 2
    n_odd = n_full % 2
    tail = per_tile - n_full * CHUNK
    assert tail % LANES == 0
    mesh = plsc.VectorSubcoreMesh(core_axis_name="c", subcore_axis_name="s")

    scratch = [
        pltpu.VMEM((CHUNK,), jnp.int32),
        pltpu.VMEM((CHUNK,), jnp.int32),
        pltpu.VMEM((CHUNK,), jnp.int32),
        pltpu.VMEM((CHUNK,), jnp.int32),
        pltpu.VMEM((CHUNK, D), jnp.float32),
        pltpu.VMEM((CHUNK, D), jnp.float32),
        pltpu.VMEM((ROWS_PER_TILE // 8, D), jnp.float32),
        pltpu.VMEM_SHARED((SEG_PAD, D), jnp.float32),
        pltpu.SemaphoreType.DMA,
        pltpu.SemaphoreType.DMA,
        pltpu.SemaphoreType.DMA,
        pltpu.SemaphoreType.DMA,
    ]
    if tail:
        scratch += [
            pltpu.VMEM((tail,), jnp.int32),
            pltpu.VMEM((tail,), jnp.int32),
            pltpu.VMEM((tail, D), jnp.float32),
        ]

    @functools.partial(
        pl.kernel,
        mesh=mesh,
        out_type=jax.ShapeDtypeStruct((N_CORES, SEG_PAD, D), jnp.float32),
        scratch_types=scratch,
    )
    def k(x_hbm, s1_hbm, d1_hbm, s2_hbm, d2_hbm, y_hbm, *refs):
        if tail:
            (idx_s0, idx_s1, idx_d0, idx_d1, rows0, rows1, compact, acc,
             gsem0, gsem1, asem0, asem1, tidx_s, tidx_d, trows) = refs
        else:
            (idx_s0, idx_s1, idx_d0, idx_d1, rows0, rows1, compact, acc,
             gsem0, gsem1, asem0, asem1) = refs
        cid = lax.axis_index("c")
        sid = lax.axis_index("s")
        row_base = sid * ROWS_PER_TILE
        edge_base = sid * per_tile

        zero16 = jnp.zeros((LANES,), jnp.float32)
        one16 = jnp.ones((LANES,), jnp.float32)
        iota16 = lax.iota(jnp.int32, 16)

        def fill_idx(rb):
            for i in range(CHUNK // LANES):
                idx_d0[pl.ds(i * LANES, LANES)] = rb + i * LANES + iota16

        def fill_buf(buf, val, nrows=CHUNK):
            def body(r, carry):
                for j in range(D // LANES):
                    buf[r, pl.ds(j * LANES, LANES)] = val
                return carry

            lax.fori_loop(0, nrows, body, 0)

        fill_buf(rows0, zero16)

        for kk in range(ROW_CHUNKS):
            fill_idx(row_base + kk * CHUNK)
            pltpu.sync_copy(rows0, acc.at[idx_d0])

        fill_buf(rows0, one16)
        if tail:
            fill_buf(trows, one16, nrows=tail)

        plsc.subcore_barrier()

        def pick_etype(fn):
            @pl.when(cid == 0)
            def _():
                fn(0)

            @pl.when(cid == 1)
            def _():
                fn(1)

        def count_pass(etype):
            dst_hbm = d1_hbm if etype == 0 else d2_hbm

            def step(k2, carry):
                base = edge_base + (2 * k2) * CHUNK
                pltpu.sync_copy(dst_hbm.at[pl.ds(base, CHUNK)], idx_d0)
                a0 = pltpu.async_copy(rows0, acc.at[idx_d0], asem0, add=True)
                pltpu.sync_copy(dst_hbm.at[pl.ds(base + CHUNK, CHUNK)], idx_d1)
                a1 = pltpu.async_copy(rows0, acc.at[idx_d1], asem1, add=True)
                a0.wait()
                a1.wait()
                return carry

            lax.fori_loop(0, n_pair, step, 0)
            for _extra in range(n_odd):
                base = edge_base + (2 * n_pair) * CHUNK
                pltpu.sync_copy(dst_hbm.at[pl.ds(base, CHUNK)], idx_d0)
                pltpu.sync_copy(rows0, acc.at[idx_d0], add=True)
            if tail:
                base = edge_base + n_full * CHUNK
                pltpu.sync_copy(dst_hbm.at[pl.ds(base, tail)], tidx_d)
                pltpu.sync_copy(trows, acc.at[tidx_d], add=True)

        pick_etype(count_pass)

        plsc.subcore_barrier()

        fill_buf(rows1, zero16)

        def dchunk_cnt(kk, carry):
            fill_idx(row_base + kk * CHUNK)
            pltpu.sync_copy(acc.at[idx_d0], rows0)
            for r in range(CHUNK):
                compact[kk * (CHUNK // 8) + r // 8,
                        pl.ds((r % 8) * LANES, LANES)] = rows0[r, pl.ds(0, LANES)]
            pltpu.sync_copy(rows1, acc.at[idx_d0])
            return carry

        lax.fori_loop(0, ROW_CHUNKS, dchunk_cnt, 0)

        plsc.subcore_barrier()

        def sum_pass(etype):
            src_hbm = s1_hbm if etype == 0 else s2_hbm
            dst_hbm = d1_hbm if etype == 0 else d2_hbm

            def step(k2, carry):
                base = edge_base + (2 * k2) * CHUNK
                pltpu.sync_copy(src_hbm.at[pl.ds(base, CHUNK)], idx_s0)
                g0 = pltpu.async_copy(x_hbm.at[idx_s0], rows0, gsem0)
                pltpu.sync_copy(src_hbm.at[pl.ds(base + CHUNK, CHUNK)], idx_s1)
                g1 = pltpu.async_copy(x_hbm.at[idx_s1], rows1, gsem1)
                pltpu.sync_copy(dst_hbm.at[pl.ds(base, CHUNK)], idx_d0)
                pltpu.sync_copy(dst_hbm.at[pl.ds(base + CHUNK, CHUNK)], idx_d1)
                g0.wait()
                a0 = pltpu.async_copy(rows0, acc.at[idx_d0], asem0, add=True)
                g1.wait()
                a1 = pltpu.async_copy(rows1, acc.at[idx_d1], asem1, add=True)
                a0.wait()
                a1.wait()
                return carry

            lax.fori_loop(0, n_pair, step, 0)
            for _extra in range(n_odd):
                base = edge_base + (2 * n_pair) * CHUNK
                pltpu.sync_copy(src_hbm.at[pl.ds(base, CHUNK)], idx_s0)
                pltpu.sync_copy(dst_hbm.at[pl.ds(base, CHUNK)], idx_d0)
                pltpu.async_copy(x_hbm.at[idx_s0], rows0, gsem0).wait()
                pltpu.sync_copy(rows0, acc.at[idx_d0], add=True)
            if tail:
                base = edge_base + n_full * CHUNK
                pltpu.sync_copy(src_hbm.at[pl.ds(base, tail)], tidx_s)
                pltpu.sync_copy(dst_hbm.at[pl.ds(base, tail)], tidx_d)
                pltpu.async_copy(x_hbm.at[tidx_s], trows, gsem0).wait()
                pltpu.sync_copy(trows, acc.at[tidx_d], add=True)

        pick_etype(sum_pass)

        plsc.subcore_barrier()

        def writeback(etype):
            def dchunk(kk, carry):
                rb = row_base + kk * CHUNK
                fill_idx(rb)
                pltpu.sync_copy(acc.at[idx_d0], rows0)
                for r in range(CHUNK):
                    cnt16 = compact[kk * (CHUNK // 8) + r // 8,
                                    pl.ds((r % 8) * LANES, LANES)]
                    scale = 0.5 / jnp.maximum(cnt16, 1.0)
                    for j in range(D // LANES):
                        sl = pl.ds(j * LANES, LANES)
                        rows0[r, sl] = rows0[r, sl] * scale
                pltpu.sync_copy(rows0, y_hbm.at[etype, pl.ds(rb, CHUNK)])
                return carry

            lax.fori_loop(0, ROW_CHUNKS, dchunk, 0)

        pick_etype(writeback)

    return k(x_func, s1, d1, s2, d2)


def _combine_body(y_ref, x_ref, o_ref):
    o_ref[0, :, :] = y_ref[0, :, :] + y_ref[1, :, :]
    o_ref[1, :, :] = x_ref[:, :]


def _combine(y, x_func, n_seg):
    blk = 1000
    grid = n_seg // blk
    return pl.pallas_call(
        _combine_body,
        grid=(grid,),
        in_specs=[
            pl.BlockSpec((N_CORES, blk, D), lambda i: (0, i, 0)),
            pl.BlockSpec((blk, D), lambda i: (i, 0)),
        ],
        out_specs=pl.BlockSpec((N_CORES, blk, D), lambda i: (0, i, 0)),
        out_shape=jax.ShapeDtypeStruct((N_CORES, n_seg, D), jnp.float32),
    )(y, x_func)


def kernel(x_vul, x_func, edge_index_1, edge_index_2):
    n_seg = x_vul.shape[0]
    n_edges = edge_index_1.shape[1]
    assert n_edges % (N_SUB * LANES) == 0
    per_tile = n_edges // N_SUB

    s1 = edge_index_1[0].astype(jnp.int32)
    d1 = edge_index_1[1].astype(jnp.int32)
    s2 = edge_index_2[0].astype(jnp.int32)
    d2 = edge_index_2[1].astype(jnp.int32)

    y = _sc_segment_partials(x_func, s1, d1, s2, d2, per_tile)
    return _combine(y, x_func, n_seg)


# speedup vs baseline: 6.5563x; 1.0183x over previous
import functools

import jax
import jax.numpy as jnp
from jax import lax
from jax.experimental import pallas as pl
from jax.experimental.pallas import tpu as pltpu
from jax.experimental.pallas import tpu_sc as plsc

D = 128
LANES = 16
N_SUB = 16
N_CORES = 2
CHUNK = 128
SEG_PAD = 10240
ROWS_PER_TILE = SEG_PAD // N_SUB
ROW_CHUNKS = ROWS_PER_TILE // CHUNK


def _sc_segment_partials(x_func, s1, d1, s2, d2, per_tile):
    n_full = per_tile // CHUNK
    n_pair = n_full // 2
    n_odd = n_full % 2
    tail = per_tile - n_full * CHUNK
    assert tail % LANES == 0
    mesh = plsc.VectorSubcoreMesh(core_axis_name="c", subcore_axis_name="s")

    scratch = [
        pltpu.VMEM((CHUNK,), jnp.int32),
        pltpu.VMEM((CHUNK,), jnp.int32),
        pltpu.VMEM((CHUNK,), jnp.int32),
        pltpu.VMEM((CHUNK,), jnp.int32),
        pltpu.VMEM((CHUNK, D), jnp.float32),
        pltpu.VMEM((CHUNK, D), jnp.float32),
        pltpu.VMEM((ROWS_PER_TILE // 8, D), jnp.float32),
        pltpu.VMEM_SHARED((SEG_PAD, D), jnp.float32),
        pltpu.SemaphoreType.DMA,
        pltpu.SemaphoreType.DMA,
        pltpu.SemaphoreType.DMA,
        pltpu.SemaphoreType.DMA,
    ]
    if tail:
        scratch += [
            pltpu.VMEM((tail,), jnp.int32),
            pltpu.VMEM((tail,), jnp.int32),
            pltpu.VMEM((tail, D), jnp.float32),
        ]

    @functools.partial(
        pl.kernel,
        mesh=mesh,
        out_type=jax.ShapeDtypeStruct((N_CORES, SEG_PAD, D), jnp.float32),
        scratch_types=scratch,
    )
    def k(x_hbm, s1_hbm, d1_hbm, s2_hbm, d2_hbm, y_hbm, *refs):
        if tail:
            (idx_s0, idx_s1, idx_d0, idx_d1, rows0, rows1, compact, acc,
             gsem0, gsem1, asem0, asem1, tidx_s, tidx_d, trows) = refs
        else:
            (idx_s0, idx_s1, idx_d0, idx_d1, rows0, rows1, compact, acc,
             gsem0, gsem1, asem0, asem1) = refs
        cid = lax.axis_index("c")
        sid = lax.axis_index("s")
        row_base = sid * ROWS_PER_TILE
        edge_base = sid * per_tile

        zero16 = jnp.zeros((LANES,), jnp.float32)
        one16 = jnp.ones((LANES,), jnp.float32)
        iota16 = lax.iota(jnp.int32, 16)

        def fill_idx(rb):
            for i in range(CHUNK // LANES):
                idx_d0[pl.ds(i * LANES, LANES)] = rb + i * LANES + iota16

        def fill_buf(buf, val, nrows=CHUNK):
            def body(r, carry):
                for j in range(D // LANES):
                    buf[r, pl.ds(j * LANES, LANES)] = val
                return carry

            lax.fori_loop(0, nrows, body, 0)

        fill_buf(rows0, zero16)

        for kk in range(ROW_CHUNKS):
            fill_idx(row_base + kk * CHUNK)
            pltpu.sync_copy(rows0, acc.at[idx_d0])

        fill_buf(rows0, one16)
        if tail:
            fill_buf(trows, one16, nrows=tail)

        plsc.subcore_barrier()

        def pick_etype(fn):
            @pl.when(cid == 0)
            def _():
                fn(0)

            @pl.when(cid == 1)
            def _():
                fn(1)

        def count_pass(etype):
            dst_hbm = d1_hbm if etype == 0 else d2_hbm
            bufs = (idx_d0, idx_d1, idx_s0, idx_s1)
            sems = (asem0, asem1, gsem0, gsem1)
            n_quad = n_full // 4
            n_rem = n_full % 4

            def step(k4, carry):
                base = edge_base + (4 * k4) * CHUNK
                pend = []
                for q in range(4):
                    pltpu.sync_copy(
                        dst_hbm.at[pl.ds(base + q * CHUNK, CHUNK)], bufs[q])
                    pend.append(pltpu.async_copy(
                        rows0, acc.at[bufs[q]], sems[q], add=True))
                for a in pend:
                    a.wait()
                return carry

            lax.fori_loop(0, n_quad, step, 0)
            for q in range(n_rem):
                base = edge_base + (4 * n_quad + q) * CHUNK
                pltpu.sync_copy(dst_hbm.at[pl.ds(base, CHUNK)], bufs[q])
                pltpu.sync_copy(rows0, acc.at[bufs[q]], add=True)
            if tail:
                base = edge_base + n_full * CHUNK
                pltpu.sync_copy(dst_hbm.at[pl.ds(base, tail)], tidx_d)
                pltpu.sync_copy(trows, acc.at[tidx_d], add=True)

        pick_etype(count_pass)

        plsc.subcore_barrier()

        fill_buf(rows1, zero16)

        def dchunk_cnt(kk, carry):
            fill_idx(row_base + kk * CHUNK)
            pltpu.sync_copy(acc.at[idx_d0], rows0)
            for r in range(CHUNK):
                compact[kk * (CHUNK // 8) + r // 8,
                        pl.ds((r % 8) * LANES, LANES)] = rows0[r, pl.ds(0, LANES)]
            pltpu.sync_copy(rows1, acc.at[idx_d0])
            return carry

        lax.fori_loop(0, ROW_CHUNKS, dchunk_cnt, 0)

        plsc.subcore_barrier()

        def sum_pass(etype):
            src_hbm = s1_hbm if etype == 0 else s2_hbm
            dst_hbm = d1_hbm if etype == 0 else d2_hbm

            def step(k2, carry):
                base = edge_base + (2 * k2) * CHUNK
                pltpu.sync_copy(src_hbm.at[pl.ds(base, CHUNK)], idx_s0)
                g0 = pltpu.async_copy(x_hbm.at[idx_s0], rows0, gsem0)
                pltpu.sync_copy(src_hbm.at[pl.ds(base + CHUNK, CHUNK)], idx_s1)
                g1 = pltpu.async_copy(x_hbm.at[idx_s1], rows1, gsem1)
                pltpu.sync_copy(dst_hbm.at[pl.ds(base, CHUNK)], idx_d0)
                pltpu.sync_copy(dst_hbm.at[pl.ds(base + CHUNK, CHUNK)], idx_d1)
                g0.wait()
                a0 = pltpu.async_copy(rows0, acc.at[idx_d0], asem0, add=True)
                g1.wait()
                a1 = pltpu.async_copy(rows1, acc.at[idx_d1], asem1, add=True)
                a0.wait()
                a1.wait()
                return carry

            lax.fori_loop(0, n_pair, step, 0)
            for _extra in range(n_odd):
                base = edge_base + (2 * n_pair) * CHUNK
                pltpu.sync_copy(src_hbm.at[pl.ds(base, CHUNK)], idx_s0)
                pltpu.sync_copy(dst_hbm.at[pl.ds(base, CHUNK)], idx_d0)
                pltpu.async_copy(x_hbm.at[idx_s0], rows0, gsem0).wait()
                pltpu.sync_copy(rows0, acc.at[idx_d0], add=True)
            if tail:
                base = edge_base + n_full * CHUNK
                pltpu.sync_copy(src_hbm.at[pl.ds(base, tail)], tidx_s)
                pltpu.sync_copy(dst_hbm.at[pl.ds(base, tail)], tidx_d)
                pltpu.async_copy(x_hbm.at[tidx_s], trows, gsem0).wait()
                pltpu.sync_copy(trows, acc.at[tidx_d], add=True)

        pick_etype(sum_pass)

        plsc.subcore_barrier()

        def writeback(etype):
            def dchunk(kk, carry):
                rb = row_base + kk * CHUNK
                fill_idx(rb)
                pltpu.sync_copy(acc.at[idx_d0], rows0)
                for r in range(CHUNK):
                    cnt16 = compact[kk * (CHUNK // 8) + r // 8,
                                    pl.ds((r % 8) * LANES, LANES)]
                    scale = 0.5 / jnp.maximum(cnt16, 1.0)
                    for j in range(D // LANES):
                        sl = pl.ds(j * LANES, LANES)
                        rows0[r, sl] = rows0[r, sl] * scale
                pltpu.sync_copy(rows0, y_hbm.at[etype, pl.ds(rb, CHUNK)])
                return carry

            lax.fori_loop(0, ROW_CHUNKS, dchunk, 0)

        pick_etype(writeback)

    return k(x_func, s1, d1, s2, d2)


def _combine_body(y_ref, x_ref, o_ref):
    o_ref[0, :, :] = y_ref[0, :, :] + y_ref[1, :, :]
    o_ref[1, :, :] = x_ref[:, :]


def _combine(y, x_func, n_seg):
    blk = 1000
    grid = n_seg // blk
    return pl.pallas_call(
        _combine_body,
        grid=(grid,),
        in_specs=[
            pl.BlockSpec((N_CORES, blk, D), lambda i: (0, i, 0)),
            pl.BlockSpec((blk, D), lambda i: (i, 0)),
        ],
        out_specs=pl.BlockSpec((N_CORES, blk, D), lambda i: (0, i, 0)),
        out_shape=jax.ShapeDtypeStruct((N_CORES, n_seg, D), jnp.float32),
    )(y, x_func)


def kernel(x_vul, x_func, edge_index_1, edge_index_2):
    n_seg = x_vul.shape[0]
    n_edges = edge_index_1.shape[1]
    assert n_edges % (N_SUB * LANES) == 0
    per_tile = n_edges // N_SUB

    s1 = edge_index_1[0].astype(jnp.int32)
    d1 = edge_index_1[1].astype(jnp.int32)
    s2 = edge_index_2[0].astype(jnp.int32)
    d2 = edge_index_2[1].astype(jnp.int32)

    y = _sc_segment_partials(x_func, s1, d1, s2, d2, per_tile)
    return _combine(y, x_func, n_seg)


# speedup vs baseline: 6.5995x; 1.0066x over previous
import functools

import jax
import jax.numpy as jnp
from jax import lax
from jax.experimental import pallas as pl
from jax.experimental.pallas import tpu as pltpu
from jax.experimental.pallas import tpu_sc as plsc

D = 128
LANES = 16
N_SUB = 16
N_CORES = 2
CHUNK = 128
SEG_PAD = 10240
ROWS_PER_TILE = SEG_PAD // N_SUB
ROW_CHUNKS = ROWS_PER_TILE // CHUNK


def _sc_segment_partials(x_func, s1, d1, s2, d2, per_tile):
    n_full = per_tile // CHUNK
    n_pair = n_full // 2
    n_odd = n_full % 2
    tail = per_tile - n_full * CHUNK
    assert tail % LANES == 0
    mesh = plsc.VectorSubcoreMesh(core_axis_name="c", subcore_axis_name="s")

    scratch = [
        pltpu.VMEM((CHUNK,), jnp.int32),
        pltpu.VMEM((CHUNK,), jnp.int32),
        pltpu.VMEM((CHUNK,), jnp.int32),
        pltpu.VMEM((CHUNK,), jnp.int32),
        pltpu.VMEM((CHUNK, D), jnp.float32),
        pltpu.VMEM((CHUNK, D), jnp.float32),
        pltpu.VMEM((ROWS_PER_TILE // 8, D), jnp.float32),
        pltpu.VMEM_SHARED((SEG_PAD, D), jnp.float32),
        pltpu.SemaphoreType.DMA,
        pltpu.SemaphoreType.DMA,
        pltpu.SemaphoreType.DMA,
        pltpu.SemaphoreType.DMA,
    ]
    if tail:
        scratch += [
            pltpu.VMEM((tail,), jnp.int32),
            pltpu.VMEM((tail,), jnp.int32),
            pltpu.VMEM((tail, D), jnp.float32),
        ]

    @functools.partial(
        pl.kernel,
        mesh=mesh,
        out_type=jax.ShapeDtypeStruct((N_CORES, SEG_PAD, D), jnp.float32),
        scratch_types=scratch,
    )
    def k(x_hbm, s1_hbm, d1_hbm, s2_hbm, d2_hbm, y_hbm, *refs):
        if tail:
            (idx_s0, idx_s1, idx_d0, idx_d1, rows0, rows1, compact, acc,
             gsem0, gsem1, asem0, asem1, tidx_s, tidx_d, trows) = refs
        else:
            (idx_s0, idx_s1, idx_d0, idx_d1, rows0, rows1, compact, acc,
             gsem0, gsem1, asem0, asem1) = refs
        cid = lax.axis_index("c")
        sid = lax.axis_index("s")
        row_base = sid * ROWS_PER_TILE
        edge_base = sid * per_tile

        zero16 = jnp.zeros((LANES,), jnp.float32)
        one16 = jnp.ones((LANES,), jnp.float32)
        iota16 = lax.iota(jnp.int32, 16)

        def fill_idx(rb):
            for i in range(CHUNK // LANES):
                idx_d0[pl.ds(i * LANES, LANES)] = rb + i * LANES + iota16

        def fill_buf(buf, val, nrows=CHUNK):
            def body(r, carry):
                for j in range(D // LANES):
                    buf[r, pl.ds(j * LANES, LANES)] = val
                return carry

            lax.fori_loop(0, nrows, body, 0)

        fill_buf(rows0, zero16)

        for kk in range(ROW_CHUNKS):
            fill_idx(row_base + kk * CHUNK)
            pltpu.sync_copy(rows0, acc.at[idx_d0])

        fill_buf(rows0, one16)
        if tail:
            fill_buf(trows, one16, nrows=tail)

        plsc.subcore_barrier()

        def pick_etype(fn):
            @pl.when(cid == 0)
            def _():
                fn(0)

            @pl.when(cid == 1)
            def _():
                fn(1)

        def count_pass(etype):
            dst_hbm = d1_hbm if etype == 0 else d2_hbm
            bufs = (idx_d0, idx_d1, idx_s0, idx_s1)
            sems = (asem0, asem1, gsem0, gsem1)
            n_quad = n_full // 4
            n_rem = n_full % 4

            def step(k4, carry):
                base = edge_base + (4 * k4) * CHUNK
                pend = []
                for q in range(4):
                    pltpu.sync_copy(
                        dst_hbm.at[pl.ds(base + q * CHUNK, CHUNK)], bufs[q])
                    pend.append(pltpu.async_copy(
                        rows0, acc.at[bufs[q]], sems[q], add=True))
                for a in pend:
                    a.wait()
                return carry

            lax.fori_loop(0, n_quad, step, 0)
            for q in range(n_rem):
                base = edge_base + (4 * n_quad + q) * CHUNK
                pltpu.sync_copy(dst_hbm.at[pl.ds(base, CHUNK)], bufs[q])
                pltpu.sync_copy(rows0, acc.at[bufs[q]], add=True)
            if tail:
                base = edge_base + n_full * CHUNK
                pltpu.sync_copy(dst_hbm.at[pl.ds(base, tail)], tidx_d)
                pltpu.sync_copy(trows, acc.at[tidx_d], add=True)

        pick_etype(count_pass)

        plsc.subcore_barrier()

        fill_buf(rows1, zero16)

        def dchunk_cnt(kk, carry):
            fill_idx(row_base + kk * CHUNK)
            pltpu.sync_copy(acc.at[idx_d0], rows0)
            for r in range(CHUNK):
                compact[kk * (CHUNK // 8) + r // 8,
                        pl.ds((r % 8) * LANES, LANES)] = rows0[r, pl.ds(0, LANES)]
            pltpu.sync_copy(rows1, acc.at[idx_d0])
            return carry

        lax.fori_loop(0, ROW_CHUNKS, dchunk_cnt, 0)

        plsc.subcore_barrier()

        def sum_pass(etype):
            src_hbm = s1_hbm if etype == 0 else s2_hbm
            dst_hbm = d1_hbm if etype == 0 else d2_hbm

            def step(k2, carry):
                base = edge_base + (2 * k2) * CHUNK
                pltpu.sync_copy(src_hbm.at[pl.ds(base, CHUNK)], idx_s0)
                g0 = pltpu.async_copy(x_hbm.at[idx_s0], rows0, gsem0)
                pltpu.sync_copy(src_hbm.at[pl.ds(base + CHUNK, CHUNK)], idx_s1)
                g1 = pltpu.async_copy(x_hbm.at[idx_s1], rows1, gsem1)
                pltpu.sync_copy(dst_hbm.at[pl.ds(base, CHUNK)], idx_d0)
                pltpu.sync_copy(dst_hbm.at[pl.ds(base + CHUNK, CHUNK)], idx_d1)
                g0.wait()
                a0 = pltpu.async_copy(rows0, acc.at[idx_d0], asem0, add=True)
                g1.wait()
                a1 = pltpu.async_copy(rows1, acc.at[idx_d1], asem1, add=True)
                a0.wait()
                a1.wait()
                return carry

            lax.fori_loop(0, n_pair, step, 0)
            for _extra in range(n_odd):
                base = edge_base + (2 * n_pair) * CHUNK
                pltpu.sync_copy(src_hbm.at[pl.ds(base, CHUNK)], idx_s0)
                pltpu.sync_copy(dst_hbm.at[pl.ds(base, CHUNK)], idx_d0)
                pltpu.async_copy(x_hbm.at[idx_s0], rows0, gsem0).wait()
                pltpu.sync_copy(rows0, acc.at[idx_d0], add=True)
            if tail:
                base = edge_base + n_full * CHUNK
                pltpu.sync_copy(src_hbm.at[pl.ds(base, tail)], tidx_s)
                pltpu.sync_copy(dst_hbm.at[pl.ds(base, tail)], tidx_d)
                pltpu.async_copy(x_hbm.at[tidx_s], trows, gsem0).wait()
                pltpu.sync_copy(trows, acc.at[tidx_d], add=True)

        pick_etype(sum_pass)

        plsc.subcore_barrier()

        def writeback(etype):
            def dchunk(kk, carry):
                rb = row_base + kk * CHUNK
                fill_idx(rb)
                pltpu.sync_copy(acc.at[idx_d0], rows0)
                for r in range(CHUNK):
                    cnt16 = compact[kk * (CHUNK // 8) + r // 8,
                                    pl.ds((r % 8) * LANES, LANES)]
                    scale = 0.5 / jnp.maximum(cnt16, 1.0)
                    for j in range(D // LANES):
                        sl = pl.ds(j * LANES, LANES)
                        rows0[r, sl] = rows0[r, sl] * scale
                pltpu.sync_copy(rows0, y_hbm.at[etype, pl.ds(rb, CHUNK)])
                return carry

            lax.fori_loop(0, ROW_CHUNKS, dchunk, 0)

        pick_etype(writeback)

    return k(x_func, s1, d1, s2, d2)


def _make_combine_body(n_seg):
    def body(y_ref, x_ref, o_ref):
        o_ref[0, :, :] = y_ref[0, :n_seg, :] + y_ref[1, :n_seg, :]
        o_ref[1, :, :] = x_ref[:, :]

    return body


def _combine(y, x_func, n_seg):
    return pl.pallas_call(
        _make_combine_body(n_seg),
        out_shape=jax.ShapeDtypeStruct((N_CORES, n_seg, D), jnp.float32),
    )(y, x_func)


def kernel(x_vul, x_func, edge_index_1, edge_index_2):
    n_seg = x_vul.shape[0]
    n_edges = edge_index_1.shape[1]
    assert n_edges % (N_SUB * LANES) == 0
    per_tile = n_edges // N_SUB

    s1 = edge_index_1[0].astype(jnp.int32)
    d1 = edge_index_1[1].astype(jnp.int32)
    s2 = edge_index_2[0].astype(jnp.int32)
    d2 = edge_index_2[1].astype(jnp.int32)

    y = _sc_segment_partials(x_func, s1, d1, s2, d2, per_tile)
    return _combine(y, x_func, n_seg)
